# Initial kernel scaffold; baseline (speedup 1.0000x reference)
import jax
import jax.numpy as jnp
from jax.experimental import pallas as pl


def kernel(token_idxs, table, W, b):
    raise NotImplementedError("write your pallas kernel here")


# speedup vs baseline: 2.4536x; 2.4536x over previous
import functools

import jax
import jax.numpy as jnp
from jax import lax
from jax.experimental import pallas as pl
from jax.experimental.pallas import tpu as pltpu
from jax.experimental.pallas import tpu_sc as plsc

VOCAB = 1000000
DIM = 64
BATCH = 16384
BAG = 50

NC = 2
NS = 16
L = 16
NW = NC * NS
BPW = BATCH // NW
GROUP = 16
NGROUP = BPW // GROUP
ROWS = GROUP * BAG
SUB = 80
NSUB = ROWS // SUB


def _sc_body(idx_hbm, table_hbm, out_hbm, idx_v, rows_v, sums_v, sem):
    wid = lax.axis_index("s") * NC + lax.axis_index("c")
    base_bag = wid * BPW

    def group_body(g, carry):
        bag0 = base_bag + g * GROUP
        pltpu.sync_copy(idx_hbm.at[pl.ds(bag0 * BAG, ROWS)], idx_v)
        copies = []
        for s in range(NSUB):
            copies.append(
                pltpu.async_copy(
                    table_hbm.at[idx_v.at[pl.ds(s * SUB, SUB)]],
                    rows_v.at[pl.ds(s * SUB, SUB)],
                    sem,
                )
            )
        for c in copies:
            c.wait()
        for i in range(GROUP):
            row0 = i * BAG

            def bag_body(j, accs):
                base = row0 + j * 10
                for u in range(10):
                    accs = tuple(
                        accs[k] + rows_v[base + u, pl.ds(k * L, L)]
                        for k in range(DIM // L)
                    )
                return accs

            accs = lax.fori_loop(
                0, BAG // 10, bag_body,
                tuple(jnp.zeros((L,), jnp.float32) for _ in range(DIM // L)),
            )
            for k in range(DIM // L):
                sums_v[i, pl.ds(k * L, L)] = accs[k]
        pltpu.sync_copy(sums_v, out_hbm.at[pl.ds(bag0, GROUP)])
        return carry

    lax.fori_loop(0, NGROUP, group_body, 0)


def _sc_bag_sum(flat_idx, table):
    mesh = plsc.VectorSubcoreMesh(core_axis_name="c", subcore_axis_name="s")
    f = functools.partial(
        pl.kernel,
        mesh=mesh,
        compiler_params=pltpu.CompilerParams(use_tc_tiling_on_sc=False),
        out_type=jax.ShapeDtypeStruct((BATCH, DIM), jnp.float32),
        scratch_types=[
            pltpu.VMEM((ROWS,), jnp.int32),
            pltpu.VMEM((ROWS, DIM), jnp.float32),
            pltpu.VMEM((GROUP, DIM), jnp.float32),
            pltpu.SemaphoreType.DMA,
        ],
    )(_sc_body)
    return f(flat_idx, table)


def _tc_body(x_ref, w_ref, b_ref, o_ref):
    x = x_ref[...]
    n2 = jnp.sum(x * x, axis=1, keepdims=True)
    y = x * lax.rsqrt(n2)
    o_ref[...] = (
        lax.dot_general(y, w_ref[...], (((1,), (1,)), ((), ())),
                        preferred_element_type=jnp.float32)
        + b_ref[...]
    )


def _tc_norm_linear(sums, W, b2d):
    bs = 1024
    return pl.pallas_call(
        _tc_body,
        grid=(BATCH // bs,),
        in_specs=[
            pl.BlockSpec((bs, DIM), lambda i: (i, 0)),
            pl.BlockSpec((DIM, DIM), lambda i: (0, 0)),
            pl.BlockSpec((1, DIM), lambda i: (0, 0)),
        ],
        out_specs=pl.BlockSpec((bs, DIM), lambda i: (i, 0)),
        out_shape=jax.ShapeDtypeStruct((BATCH, DIM), jnp.float32),
    )(sums, W, b2d)


def kernel(token_idxs, table, W, b):
    flat_idx = token_idxs.reshape(-1).astype(jnp.int32)
    sums = _sc_bag_sum(flat_idx, table)
    return _tc_norm_linear(sums, W, b.reshape(1, DIM))
